# baseline (speedup 1.0000x reference)
import functools

import jax
import jax.numpy as jnp
from jax import lax
from jax.experimental import pallas as pl
from jax.experimental.pallas import tpu as pltpu
from jax.experimental.pallas import tpu_sc as plsc


_PREC = lax.Precision.DEFAULT


def _sc_gather(table, idx3d, n_workers, b_per_w, n_chunks, chunk):
    D = table.shape[1]
    B = n_workers * b_per_w
    mesh = plsc.VectorSubcoreMesh(core_axis_name="c", subcore_axis_name="s")

    @functools.partial(
        pl.kernel,
        mesh=mesh,
        out_type=jax.ShapeDtypeStruct((B, D), jnp.float32),
        scratch_types=[
            pltpu.VMEM((n_chunks, chunk), jnp.int32),
            pltpu.VMEM((b_per_w, D), jnp.float32),
            pltpu.SemaphoreType.DMA((n_chunks,)),
            pltpu.SemaphoreType.DMA((n_chunks,)),
        ],
    )
    def k(table_hbm, idx_hbm, out_hbm, idx_v, rows_v, gsem, wsem):
        nc = lax.axis_size("c")
        wid = lax.axis_index("s") * nc + lax.axis_index("c")
        base = wid * b_per_w
        pltpu.sync_copy(idx_hbm.at[wid], idx_v)
        gathers = [
            pltpu.make_async_copy(
                table_hbm.at[idx_v.at[j]],
                rows_v.at[pl.ds(j * chunk, chunk)],
                gsem.at[j],
            )
            for j in range(n_chunks)
        ]
        writes = [
            pltpu.make_async_copy(
                rows_v.at[pl.ds(j * chunk, chunk)],
                out_hbm.at[pl.ds(base + j * chunk, chunk)],
                wsem.at[j],
            )
            for j in range(n_chunks)
        ]
        for g in gathers:
            g.start()
        for j in range(n_chunks):
            gathers[j].wait()
            writes[j].start()
        for w in writes:
            w.wait()

    return k(table, idx3d)


def _dense_tower(xcf_t, epc, w1, b1, ecms, efnf, w2, b2):
    blk = xcf_t.shape[1]
    xcf = jnp.transpose(xcf_t)
    h = jnp.dot(xcf[:, 0:3], w1, precision=_PREC) + b1
    h = jnp.where(h >= 0, h, 0.01 * h)
    acc = jnp.dot(h, w2[0:64], precision=_PREC)
    t_cms = jnp.dot(ecms, w2[64:96], precision=_PREC)
    iota4 = lax.broadcasted_iota(jnp.int32, (blk, 4), 1)
    oh_c = (xcf[:, 3:4].astype(jnp.int32) == iota4).astype(jnp.float32)
    acc += jnp.dot(oh_c, t_cms, precision=_PREC)
    t_fnf = jnp.dot(efnf, w2[96:128], precision=_PREC)
    iota5 = lax.broadcasted_iota(jnp.int32, (blk, 5), 1)
    oh_f = (xcf[:, 4:5].astype(jnp.int32) == iota5).astype(jnp.float32)
    acc += jnp.dot(oh_f, t_fnf, precision=_PREC)
    acc += jnp.dot(epc, w2[128:256], precision=_PREC)
    acc += b2
    return jnp.maximum(acc, 0.0)


def _tc_first_half(xcf, epc0, W1, b1_2d, E_cms, E_fnf, W2, b2_2d, blk, B):
    half = epc0.shape[0]
    rep = lambda i: (0, 0)

    def body(xcf_ref, epc_ref, w1_ref, b1_ref, ecms_ref, efnf_ref,
             w2_ref, b2_ref, out_ref):
        out_ref[...] = _dense_tower(
            xcf_ref[...], epc_ref[...], w1_ref[...], b1_ref[...],
            ecms_ref[...], efnf_ref[...], w2_ref[...], b2_ref[...])

    return pl.pallas_call(
        body,
        grid=(half // blk,),
        in_specs=[
            pl.BlockSpec((5, blk), lambda i: (0, i)),
            pl.BlockSpec((blk, 128), lambda i: (i, 0)),
            pl.BlockSpec((3, 64), rep),
            pl.BlockSpec((1, 64), rep),
            pl.BlockSpec((4, 32), rep),
            pl.BlockSpec((5, 32), rep),
            pl.BlockSpec((256, 64), rep),
            pl.BlockSpec((1, 64), rep),
        ],
        out_specs=pl.BlockSpec((blk, 64), lambda i: (i, 0)),
        out_shape=jax.ShapeDtypeStruct((B, 64), jnp.float32),
    )(xcf, epc0, W1, b1_2d, E_cms, E_fnf, W2, b2_2d)


def _tc_second_half_merge(xcf, epc1, y0, W1, b1_2d, E_cms, E_fnf, W2, b2_2d,
                          blk, B):
    half = epc1.shape[0]
    nh = half // blk
    rep = lambda i: (0, 0)

    def body(xcf_ref, epc_ref, y0_ref, w1_ref, b1_ref, ecms_ref, efnf_ref,
             w2_ref, b2_ref, out_ref):
        i = pl.program_id(0)

        @pl.when(i < nh)
        def _():
            out_ref[...] = y0_ref[...]

        @pl.when(i >= nh)
        def _():
            out_ref[...] = _dense_tower(
                xcf_ref[...], epc_ref[...], w1_ref[...], b1_ref[...],
                ecms_ref[...], efnf_ref[...], w2_ref[...], b2_ref[...])

    return pl.pallas_call(
        body,
        grid=(2 * nh,),
        in_specs=[
            pl.BlockSpec((5, blk), lambda i: (0, i)),
            pl.BlockSpec((blk, 128), lambda i: (jnp.maximum(i - nh, 0), 0)),
            pl.BlockSpec((blk, 64), lambda i: (jnp.minimum(i, nh - 1), 0)),
            pl.BlockSpec((3, 64), rep),
            pl.BlockSpec((1, 64), rep),
            pl.BlockSpec((4, 32), rep),
            pl.BlockSpec((5, 32), rep),
            pl.BlockSpec((256, 64), rep),
            pl.BlockSpec((1, 64), rep),
        ],
        out_specs=pl.BlockSpec((blk, 64), lambda i: (i, 0)),
        out_shape=jax.ShapeDtypeStruct((B, 64), jnp.float32),
    )(xcf, epc1, y0, W1, b1_2d, E_cms, E_fnf, W2, b2_2d)


def kernel(FN, Active, age, club_member_status, fashion_news_frequency,
           postal_code, W1, b1, E_cms, E_fnf, E_pc, W2, b2):
    B = FN.shape[0]
    info = plsc.get_sparse_core_info()
    n_workers = info.num_cores * info.num_subcores
    b_per_w = B // n_workers
    chunk = 128
    n_chunks = b_per_w // chunk
    idx3d = postal_code.reshape(n_workers, n_chunks, chunk)
    epc0 = _sc_gather(E_pc, idx3d[: n_workers // 2].reshape(
        n_workers, n_chunks // 2, chunk), n_workers, b_per_w // 2,
        n_chunks // 2, chunk)
    epc1 = _sc_gather(E_pc, idx3d[n_workers // 2:].reshape(
        n_workers, n_chunks // 2, chunk), n_workers, b_per_w // 2,
        n_chunks // 2, chunk)
    xcf = jnp.stack(
        [FN[:, 0], Active[:, 0], age[:, 0],
         club_member_status.astype(jnp.float32),
         fashion_news_frequency.astype(jnp.float32)], axis=0)
    blk = 4096
    b1_2d = b1.reshape(1, 64)
    b2_2d = b2.reshape(1, 64)
    y0 = _tc_first_half(xcf, epc0, W1, b1_2d, E_cms, E_fnf, W2, b2_2d,
                        blk, B)
    return _tc_second_half_merge(xcf, epc1, y0, W1, b1_2d, E_cms, E_fnf,
                                 W2, b2_2d, blk, B)


# speedup vs baseline: 1.0516x; 1.0516x over previous
import functools

import jax
import jax.numpy as jnp
from jax import lax
from jax.experimental import pallas as pl
from jax.experimental.pallas import tpu as pltpu
from jax.experimental.pallas import tpu_sc as plsc


_PREC = lax.Precision.DEFAULT


def _sc_gather(table, idx3d, n_workers, b_per_w, n_chunks, chunk):
    D = table.shape[1]
    B = n_workers * b_per_w
    mesh = plsc.VectorSubcoreMesh(core_axis_name="c", subcore_axis_name="s")

    @functools.partial(
        pl.kernel,
        mesh=mesh,
        out_type=jax.ShapeDtypeStruct((B, D), jnp.float32),
        scratch_types=[
            pltpu.VMEM((n_chunks, chunk), jnp.int32),
            pltpu.VMEM((b_per_w, D), jnp.float32),
            pltpu.SemaphoreType.DMA((n_chunks,)),
            pltpu.SemaphoreType.DMA((n_chunks,)),
        ],
    )
    def k(table_hbm, idx_hbm, out_hbm, idx_v, rows_v, gsem, wsem):
        nc = lax.axis_size("c")
        wid = lax.axis_index("s") * nc + lax.axis_index("c")
        base = wid * b_per_w
        pltpu.sync_copy(idx_hbm.at[wid], idx_v)
        gathers = [
            pltpu.make_async_copy(
                table_hbm.at[idx_v.at[j]],
                rows_v.at[pl.ds(j * chunk, chunk)],
                gsem.at[j],
            )
            for j in range(n_chunks)
        ]
        writes = [
            pltpu.make_async_copy(
                rows_v.at[pl.ds(j * chunk, chunk)],
                out_hbm.at[pl.ds(base + j * chunk, chunk)],
                wsem.at[j],
            )
            for j in range(n_chunks)
        ]
        for g in gathers:
            g.start()
        for j in range(n_chunks):
            gathers[j].wait()
            writes[j].start()
        for w in writes:
            w.wait()

    return k(table, idx3d)


def _tc_body(xcf_ref, epc_ref,
             w1_ref, b1_ref, ecms_ref, efnf_ref, w2_ref, b2_ref, out_ref):
    blk = xcf_ref.shape[1]
    xcf = jnp.transpose(xcf_ref[...])
    h = jnp.dot(xcf[:, 0:3], w1_ref[...], precision=_PREC) + b1_ref[...]
    h = jnp.where(h >= 0, h, 0.01 * h)
    w2 = w2_ref[...]
    acc = jnp.dot(h, w2[0:64], precision=_PREC)
    t_cms = jnp.dot(ecms_ref[...], w2[64:96], precision=_PREC)
    iota4 = lax.broadcasted_iota(jnp.int32, (blk, 4), 1)
    oh_c = (xcf[:, 3:4].astype(jnp.int32) == iota4).astype(jnp.float32)
    acc += jnp.dot(oh_c, t_cms, precision=_PREC)
    t_fnf = jnp.dot(efnf_ref[...], w2[96:128], precision=_PREC)
    iota5 = lax.broadcasted_iota(jnp.int32, (blk, 5), 1)
    oh_f = (xcf[:, 4:5].astype(jnp.int32) == iota5).astype(jnp.float32)
    acc += jnp.dot(oh_f, t_fnf, precision=_PREC)
    acc += jnp.dot(epc_ref[...], w2[128:256], precision=_PREC)
    acc += b2_ref[...]
    out_ref[...] = jnp.maximum(acc, 0.0)


def _tc_fused(xcf, epc, W1, b1_2d, E_cms, E_fnf, W2, b2_2d, blk):
    B = epc.shape[0]
    grid = (B // blk,)
    row = lambda i: (i, 0)
    rep = lambda i: (0, 0)
    return pl.pallas_call(
        _tc_body,
        grid=grid,
        in_specs=[
            pl.BlockSpec((5, blk), lambda i: (0, i)),
            pl.BlockSpec((blk, 128), row),
            pl.BlockSpec((3, 64), rep),
            pl.BlockSpec((1, 64), rep),
            pl.BlockSpec((4, 32), rep),
            pl.BlockSpec((5, 32), rep),
            pl.BlockSpec((256, 64), rep),
            pl.BlockSpec((1, 64), rep),
        ],
        out_specs=pl.BlockSpec((blk, 64), row),
        out_shape=jax.ShapeDtypeStruct((B, 64), jnp.float32),
    )(xcf, epc, W1, b1_2d, E_cms, E_fnf, W2, b2_2d)


def kernel(FN, Active, age, club_member_status, fashion_news_frequency,
           postal_code, W1, b1, E_cms, E_fnf, E_pc, W2, b2):
    B = FN.shape[0]
    info = plsc.get_sparse_core_info()
    n_workers = info.num_cores * info.num_subcores
    b_per_w = B // n_workers
    chunk = 128
    n_chunks = b_per_w // chunk
    idx3d = postal_code.reshape(n_workers, n_chunks, chunk)
    epc = _sc_gather(E_pc, idx3d, n_workers, b_per_w, n_chunks, chunk)
    xcf = jnp.stack(
        [FN[:, 0], Active[:, 0], age[:, 0],
         club_member_status.astype(jnp.float32),
         fashion_news_frequency.astype(jnp.float32)], axis=0)
    return _tc_fused(
        xcf, epc, W1, b1.reshape(1, 64), E_cms, E_fnf, W2, b2.reshape(1, 64),
        blk=4096,
    )
